# baseline (speedup 1.0000x reference)
import jax
import jax.numpy as jnp
from jax.experimental import pallas as pl
from jax.experimental.pallas import tpu as pltpu

K = 8192
D = 49
N = 4


def _vq_body(x_ref, cb_ref, out_ref):
    xs = x_ref[...]
    cb = cb_ref[...]
    b2r = jax.lax.dot_general(
        jnp.ones((1, D), jnp.float32), cb * cb, (((1,), (1,)), ((), ())),
        preferred_element_type=jnp.float32)
    dots = jax.lax.dot_general(
        xs, cb, (((1,), (1,)), ((), ())),
        preferred_element_type=jnp.float32)
    dist = b2r - 2.0 * dots
    idx = jnp.argmin(dist, axis=1)
    onehot = (jax.lax.broadcasted_iota(jnp.int32, (N, K), 1)
              == idx[:, None]).astype(jnp.float32)
    zq = jax.lax.dot_general(
        onehot, cb, (((1,), (0,)), ((), ())),
        preferred_element_type=jnp.float32)
    res = xs + (zq - xs)
    out_ref[...] = jnp.reshape(res, (N, 7, 7))


def kernel(x, codebook):
    return pl.pallas_call(
        _vq_body,
        out_shape=jax.ShapeDtypeStruct((N, 7, 7), jnp.float32),
    )(x, codebook)


# speedup vs baseline: 1.0332x; 1.0332x over previous
import jax
import jax.numpy as jnp
from jax.experimental import pallas as pl
from jax.experimental.pallas import tpu as pltpu

K = 8192
D = 49
N = 4


def _vq_body(x_ref, cb_ref, out_ref):
    xs = x_ref[...]
    cb = cb_ref[...]
    b2r = jax.lax.dot_general(
        jnp.ones((1, D), jnp.float32), cb * cb, (((1,), (1,)), ((), ())),
        preferred_element_type=jnp.float32)
    dots = jax.lax.dot_general(
        xs, cb, (((1,), (1,)), ((), ())),
        preferred_element_type=jnp.float32)
    dist = b2r - 2.0 * dots
    idx = jnp.argmin(dist, axis=1)
    rows = [cb_ref[pl.ds(idx[q], 1), :] for q in range(N)]
    zq = jnp.concatenate(rows, axis=0)
    res = xs + (zq - xs)
    out_ref[...] = jnp.reshape(res, (N, 7, 7))


def kernel(x, codebook):
    return pl.pallas_call(
        _vq_body,
        out_shape=jax.ShapeDtypeStruct((N, 7, 7), jnp.float32),
    )(x, codebook)


# speedup vs baseline: 1.0345x; 1.0012x over previous
import jax
import jax.numpy as jnp
from jax.experimental import pallas as pl
from jax.experimental.pallas import tpu as pltpu

K = 8192
D = 49
N = 4


def _vq_body(x_hbm, cb_hbm, out_ref, x_v, cb_v, sem1, sem2):
    cp1 = pltpu.make_async_copy(cb_hbm, cb_v, sem1)
    cp2 = pltpu.make_async_copy(x_hbm, x_v, sem2)
    cp1.start()
    cp2.start()
    cp2.wait()
    cp1.wait()
    xs = x_v[...]
    cb = cb_v[...]
    b2r = jax.lax.dot_general(
        jnp.ones((1, D), jnp.float32), cb * cb, (((1,), (1,)), ((), ())),
        preferred_element_type=jnp.float32)
    dots = jax.lax.dot_general(
        xs, cb, (((1,), (1,)), ((), ())),
        preferred_element_type=jnp.float32)
    dist = b2r - 2.0 * dots
    idx = jnp.argmin(dist, axis=1)
    rows = [cb_v[pl.ds(idx[q], 1), :] for q in range(N)]
    zq = jnp.concatenate(rows, axis=0)
    res = xs + (zq - xs)
    out_ref[...] = jnp.reshape(res, (N, 7, 7))


def kernel(x, codebook):
    return pl.pallas_call(
        _vq_body,
        in_specs=[
            pl.BlockSpec(memory_space=pltpu.MemorySpace.HBM),
            pl.BlockSpec(memory_space=pltpu.MemorySpace.HBM),
        ],
        out_shape=jax.ShapeDtypeStruct((N, 7, 7), jnp.float32),
        scratch_shapes=[
            pltpu.VMEM((N, D), jnp.float32),
            pltpu.VMEM((K, D), jnp.float32),
            pltpu.SemaphoreType.DMA,
            pltpu.SemaphoreType.DMA,
        ],
    )(x, codebook)
